# baseline (speedup 1.0000x reference)
import functools

import jax
import jax.numpy as jnp
from jax import lax
from jax.experimental import pallas as pl
from jax.experimental.pallas import tpu as pltpu
from jax.experimental.pallas import tpu_sc as plsc

N = 10000
E = 320000
D = 128
G = 64
EPS = 1e-5

NC = 2
NS = 16
NW = NC * NS
EW = E // NW
K = 128
NCHUNK = 80
EWP = NCHUNK * K
NP = N + 8
RPT = 632
RPT_LAST = N - (NS - 1) * RPT


def _sc_agg_body(h_hbm, src_hbm, dst_hbm, zero_hbm, out_hbm,
                 sring_v, didx_v, rows0_v, rows1_v, acc_sh,
                 sem0, sem1, ssem0, ssem1,
                 isem0, isem1, isem2, isem3):
    c = lax.axis_index("c")
    s = lax.axis_index("s")
    wid = s * NC + c
    isems = [isem0, isem1, isem2, isem3]

    row_off = pl.multiple_of(s * RPT, 8)
    cnt_tail = s == NS - 1

    def init_copy(rows, size):
        @pl.when(c == 0)
        def _():
            pltpu.async_copy(h_hbm.at[pl.ds(rows, size)],
                             acc_sh.at[pl.ds(rows, size)], sem0)

        @pl.when(c != 0)
        def _():
            pltpu.async_copy(zero_hbm.at[pl.ds(rows, size)],
                             acc_sh.at[pl.ds(rows, size)], sem0)

    @pl.when(cnt_tail)
    def _():
        @pl.when(c == 0)
        def _():
            pltpu.sync_copy(h_hbm.at[pl.ds((NS - 1) * RPT, RPT_LAST)],
                            acc_sh.at[pl.ds((NS - 1) * RPT, RPT_LAST)])

        @pl.when(c != 0)
        def _():
            pltpu.sync_copy(zero_hbm.at[pl.ds((NS - 1) * RPT, RPT_LAST)],
                            acc_sh.at[pl.ds((NS - 1) * RPT, RPT_LAST)])

        pltpu.sync_copy(zero_hbm.at[pl.ds(0, 8)], acc_sh.at[pl.ds(N, 8)])

    @pl.when(jnp.logical_not(cnt_tail))
    def _():
        init_copy(row_off, RPT)

    pltpu.async_copy(dst_hbm.at[wid], didx_v, ssem0)

    @pl.when(jnp.logical_not(cnt_tail))
    def _():
        pltpu.make_async_copy(zero_hbm.at[pl.ds(row_off, RPT)],
                              acc_sh.at[pl.ds(row_off, RPT)], sem0).wait()

    pltpu.make_async_copy(dst_hbm.at[wid], didx_v, ssem0).wait()
    plsc.subcore_barrier()

    ibase = pl.multiple_of(wid * EWP, 8)

    def fetch_idx(i, slot, sem):
        @pl.when(i < NCHUNK)
        def _():
            pltpu.async_copy(src_hbm.at[pl.ds(ibase + i * K, K)],
                             sring_v.at[slot], sem)

    def wait_idx(i, slot, sem):
        pltpu.make_async_copy(src_hbm.at[pl.ds(ibase + i * K, K)],
                              sring_v.at[slot], sem).wait()

    def gather(slot, buf, sem):
        pltpu.async_copy(h_hbm.at[sring_v.at[slot]], buf, sem)

    def wait_gather(slot, buf, sem):
        pltpu.make_async_copy(h_hbm.at[sring_v.at[slot]], buf, sem).wait()

    def scatter_add(i, buf, sem):
        pltpu.async_copy(buf, acc_sh.at[didx_v.at[i]], sem, add=True)

    def wait_scatter(i, buf, sem):
        pltpu.make_async_copy(buf, acc_sh.at[didx_v.at[i]], sem).wait()

    for p in range(4):
        fetch_idx(p, p, isems[p])
    wait_idx(0, 0, isems[0])
    gather(0, rows0_v, sem0)
    wait_idx(1, 1, isems[1])
    gather(1, rows1_v, sem1)

    bufs = [rows0_v, rows1_v]
    gsems = [sem0, sem1]
    ssems = [ssem0, ssem1]

    def quad(j, carry):
        i0 = 4 * j
        for h in range(2):
            for p in range(2):
                i = i0 + 2 * h + p
                slot = (2 * h + p) & 3
                wait_gather(slot, bufs[p], gsems[p])
                fetch_idx(i + 4, slot, isems[slot])
                scatter_add(i, bufs[p], ssems[p])
            for p in range(2):
                i = i0 + 2 * h + p + 2
                slot = (2 * h + p + 2) & 3
                wait_scatter(i - 2, bufs[p], ssems[p])

                @pl.when(i < NCHUNK)
                def _(i=i, slot=slot, p=p):
                    wait_idx(i, slot, isems[slot])
                    gather(slot, bufs[p], gsems[p])

        return carry

    lax.fori_loop(0, NCHUNK // 4, quad, 0)

    plsc.subcore_barrier()

    @pl.when(s < NS - 1)
    def _():
        pltpu.sync_copy(acc_sh.at[pl.ds(row_off, RPT)],
                        out_hbm.at[c, pl.ds(row_off, RPT)])

    @pl.when(s == NS - 1)
    def _():
        pltpu.sync_copy(acc_sh.at[pl.ds((NS - 1) * RPT, RPT_LAST)],
                        out_hbm.at[c, pl.ds((NS - 1) * RPT, RPT_LAST)])


@functools.cache
def _sc_agg_kernel():
    return pl.kernel(
        _sc_agg_body,
        out_type=jax.ShapeDtypeStruct((NC, N, D), jnp.float32),
        mesh=plsc.VectorSubcoreMesh(core_axis_name="c", subcore_axis_name="s",
                                    num_cores=NC, num_subcores=NS),
        scratch_types=[
            pltpu.VMEM((4, K), jnp.int32),
            pltpu.VMEM((NCHUNK, K), jnp.int32),
            pltpu.VMEM((K, D), jnp.float32),
            pltpu.VMEM((K, D), jnp.float32),
            pltpu.VMEM_SHARED((NP, D), jnp.float32),
            pltpu.SemaphoreType.DMA,
            pltpu.SemaphoreType.DMA,
            pltpu.SemaphoreType.DMA,
            pltpu.SemaphoreType.DMA,
            pltpu.SemaphoreType.DMA,
            pltpu.SemaphoreType.DMA,
            pltpu.SemaphoreType.DMA,
            pltpu.SemaphoreType.DMA,
        ],
    )


def _sc_agg(h, src, dst, zero):
    return _sc_agg_kernel()(h, src, dst, zero)


def _tc_layer_body(agg_ref, xl_in_ref, w1_ref, b1_ref, w2_ref, b2_ref,
                   g_ref, be_ref, batch_ref, ho_ref, xl_ref, po_ref):
    h = agg_ref[0] + agg_ref[1]
    h = jnp.dot(h, w1_ref[...], preferred_element_type=jnp.float32)
    h = h + b1_ref[...]
    h = jnp.where(h > 0, h, 0.01 * h)
    h = jnp.dot(h, w2_ref[...], preferred_element_type=jnp.float32)
    h = h + b2_ref[...]
    h = jnp.where(h > 0, h, 0.01 * h)
    m = jnp.mean(h, axis=1, keepdims=True)
    hc = h - m
    v = jnp.mean(hc * hc, axis=1, keepdims=True)
    hn = hc * lax.rsqrt(v + EPS)
    bm = jnp.mean(hn, axis=0, keepdims=True)
    hb = hn - bm
    bv = jnp.mean(hb * hb, axis=0, keepdims=True)
    hb = hb * lax.rsqrt(bv + EPS) * g_ref[...] + be_ref[...]
    ho_ref[...] = hb
    xl_ref[...] = hb
    oh = (batch_ref[...] == lax.broadcasted_iota(jnp.int32, (G, 1), 0))
    po_ref[...] = jnp.dot(oh.astype(jnp.float32), hb,
                          preferred_element_type=jnp.float32)


@functools.cache
def _tc_layer_kernel(layer):
    return pl.pallas_call(
        _tc_layer_body,
        out_shape=(jax.ShapeDtypeStruct((N, D), jnp.float32),
                   jax.ShapeDtypeStruct((N, 3 * D), jnp.float32),
                   jax.ShapeDtypeStruct((G, D), jnp.float32)),
        in_specs=(
            pl.BlockSpec((NC, N, D), lambda i: (0, 0, 0)),
            pl.BlockSpec(memory_space=pltpu.MemorySpace.HBM),
            pl.BlockSpec((D, D), lambda i: (0, 0)),
            pl.BlockSpec((1, D), lambda i: (0, 0)),
            pl.BlockSpec((D, D), lambda i: (0, 0)),
            pl.BlockSpec((1, D), lambda i: (0, 0)),
            pl.BlockSpec((1, D), lambda i: (0, 0)),
            pl.BlockSpec((1, D), lambda i: (0, 0)),
            pl.BlockSpec((1, N), lambda i: (0, 0)),
        ),
        out_specs=(
            pl.BlockSpec((N, D), lambda i: (0, 0)),
            pl.BlockSpec((N, D), lambda i, new=layer: (0, new)),
            pl.BlockSpec((G, D), lambda i: (0, 0)),
        ),
        input_output_aliases={1: 1},
        grid=(1,),
    )


def kernel(x, edge_index, batch,
           W1_0, b1_0, W2_0, b2_0, gamma_0, beta_0,
           W1_1, b1_1, W2_1, b2_1, gamma_1, beta_1,
           W1_2, b1_2, W2_2, b2_2, gamma_2, beta_2):
    src = jnp.pad(edge_index[0].reshape(NW, EW),
                  ((0, 0), (0, EWP - EW))).reshape(-1)
    dst = jnp.pad(edge_index[1].reshape(NW, EW),
                  ((0, 0), (0, EWP - EW)),
                  constant_values=N).reshape(NW, NCHUNK, K)
    zero = jnp.zeros((N, D), jnp.float32)
    batch2 = batch.reshape(1, N)
    params = [
        (W1_0, b1_0, W2_0, b2_0, gamma_0, beta_0),
        (W1_1, b1_1, W2_1, b2_1, gamma_1, beta_1),
        (W1_2, b1_2, W2_2, b2_2, gamma_2, beta_2),
    ]
    h = x
    xl = jnp.zeros((N, 3 * D), jnp.float32)
    pools = []
    for layer, (W1, b1, W2, b2, gamma, beta) in enumerate(params):
        agg = _sc_agg(h, src, dst, zero)
        h, xl, pool = _tc_layer_kernel(layer)(
            agg, xl, W1, b1.reshape(1, D), W2, b2.reshape(1, D),
            gamma.reshape(1, D), beta.reshape(1, D), batch2)
        pools.append(pool)
    x_global = jnp.concatenate(pools, axis=1)
    return (x_global, xl)


# speedup vs baseline: 2.5504x; 2.5504x over previous
import functools

import jax
import jax.numpy as jnp
from jax import lax
from jax.experimental import pallas as pl
from jax.experimental.pallas import tpu as pltpu
from jax.experimental.pallas import tpu_sc as plsc

N = 10000
E = 320000
D = 128
G = 64
EPS = 1e-5

NC = 2
NS = 16
NW = NC * NS
EW = E // NW
K = 80
NCHUNK = EW // K
RPT = 632
RPT_LAST = N - (NS - 1) * RPT


def _sc_agg_body(h_hbm, src_hbm, dst_hbm, zero_hbm, out_hbm,
                 sidx_v, didx_v, rows0_v, rows1_v, acc_sh,
                 sem0, sem1, ssem0, ssem1):
    c = lax.axis_index("c")
    s = lax.axis_index("s")
    wid = s * NC + c

    row_off = pl.multiple_of(s * RPT, 8)
    cnt_tail = s == NS - 1

    def init_copy(rows, size):
        @pl.when(c == 0)
        def _():
            pltpu.async_copy(h_hbm.at[pl.ds(rows, size)],
                             acc_sh.at[pl.ds(rows, size)], ssem1)

        @pl.when(c != 0)
        def _():
            pltpu.async_copy(zero_hbm.at[pl.ds(rows, size)],
                             acc_sh.at[pl.ds(rows, size)], ssem1)

    @pl.when(cnt_tail)
    def _():
        init_copy((NS - 1) * RPT, RPT_LAST)

    @pl.when(jnp.logical_not(cnt_tail))
    def _():
        init_copy(row_off, RPT)

    pltpu.async_copy(src_hbm.at[pl.ds(pl.multiple_of(wid * EW, 8), EW)],
                     sidx_v, sem1)
    pltpu.async_copy(dst_hbm.at[wid], didx_v, ssem0)

    def gather(i, buf, sem):
        pltpu.async_copy(h_hbm.at[sidx_v.at[pl.ds(i * K, K)]], buf, sem)

    def wait_gather(i, buf, sem):
        pltpu.make_async_copy(h_hbm.at[sidx_v.at[pl.ds(i * K, K)]],
                              buf, sem).wait()

    def scatter_add(i, buf, sem):
        pltpu.async_copy(buf, acc_sh.at[didx_v.at[i]], sem, add=True)

    def wait_scatter(i, buf, sem):
        pltpu.make_async_copy(buf, acc_sh.at[didx_v.at[i]], sem).wait()

    pltpu.make_async_copy(src_hbm.at[pl.ds(0, EW)], sidx_v, sem1).wait()
    gather(0, rows0_v, sem0)
    gather(1, rows1_v, sem1)

    @pl.when(cnt_tail)
    def _():
        pltpu.make_async_copy(zero_hbm.at[pl.ds((NS - 1) * RPT, RPT_LAST)],
                              acc_sh.at[pl.ds((NS - 1) * RPT, RPT_LAST)],
                              ssem1).wait()

    @pl.when(jnp.logical_not(cnt_tail))
    def _():
        pltpu.make_async_copy(zero_hbm.at[pl.ds(row_off, RPT)],
                              acc_sh.at[pl.ds(row_off, RPT)], ssem1).wait()

    pltpu.make_async_copy(dst_hbm.at[wid], didx_v, ssem0).wait()
    plsc.subcore_barrier()

    def pair(j, carry):
        i0 = 2 * j
        wait_gather(i0, rows0_v, sem0)
        scatter_add(i0, rows0_v, ssem0)
        wait_gather(i0 + 1, rows1_v, sem1)
        scatter_add(i0 + 1, rows1_v, ssem1)
        wait_scatter(i0, rows0_v, ssem0)
        gather(i0 + 2, rows0_v, sem0)
        wait_scatter(i0 + 1, rows1_v, ssem1)

        @pl.when(i0 + 3 < NCHUNK)
        def _():
            gather(i0 + 3, rows1_v, sem1)

        return carry

    lax.fori_loop(0, NCHUNK // 2, pair, 0)
    wait_gather(NCHUNK - 1, rows0_v, sem0)
    pltpu.sync_copy(rows0_v, acc_sh.at[didx_v.at[NCHUNK - 1]], add=True)

    plsc.subcore_barrier()

    @pl.when(s < NS - 1)
    def _():
        pltpu.sync_copy(acc_sh.at[pl.ds(row_off, RPT)],
                        out_hbm.at[c, pl.ds(row_off, RPT)])

    @pl.when(s == NS - 1)
    def _():
        pltpu.sync_copy(acc_sh.at[pl.ds((NS - 1) * RPT, RPT_LAST)],
                        out_hbm.at[c, pl.ds((NS - 1) * RPT, RPT_LAST)])


@functools.cache
def _sc_agg_kernel():
    return pl.kernel(
        _sc_agg_body,
        out_type=jax.ShapeDtypeStruct((NC, N, D), jnp.float32),
        mesh=plsc.VectorSubcoreMesh(core_axis_name="c", subcore_axis_name="s",
                                    num_cores=NC, num_subcores=NS),
        scratch_types=[
            pltpu.VMEM((EW,), jnp.int32),
            pltpu.VMEM((NCHUNK, K), jnp.int32),
            pltpu.VMEM((K, D), jnp.float32),
            pltpu.VMEM((K, D), jnp.float32),
            pltpu.VMEM_SHARED((N, D), jnp.float32),
            pltpu.SemaphoreType.DMA,
            pltpu.SemaphoreType.DMA,
            pltpu.SemaphoreType.DMA,
            pltpu.SemaphoreType.DMA,
        ],
    )


def _sc_agg(h, src, dst, zero):
    return _sc_agg_kernel()(h, src, dst, zero)


def _tc_layer_body(agg_ref, xl_in_ref, w1_ref, b1_ref, w2_ref, b2_ref,
                   g_ref, be_ref, batch_ref, ho_ref, xl_ref, po_ref):
    h = agg_ref[0] + agg_ref[1]
    h = jnp.dot(h, w1_ref[...], preferred_element_type=jnp.float32)
    h = h + b1_ref[...]
    h = jnp.where(h > 0, h, 0.01 * h)
    h = jnp.dot(h, w2_ref[...], preferred_element_type=jnp.float32)
    h = h + b2_ref[...]
    h = jnp.where(h > 0, h, 0.01 * h)
    m = jnp.mean(h, axis=1, keepdims=True)
    hc = h - m
    v = jnp.mean(hc * hc, axis=1, keepdims=True)
    hn = hc * lax.rsqrt(v + EPS)
    bm = jnp.mean(hn, axis=0, keepdims=True)
    hb = hn - bm
    bv = jnp.mean(hb * hb, axis=0, keepdims=True)
    hb = hb * lax.rsqrt(bv + EPS) * g_ref[...] + be_ref[...]
    ho_ref[...] = hb
    xl_ref[...] = hb
    oh = (batch_ref[...] == lax.broadcasted_iota(jnp.int32, (G, 1), 0))
    po_ref[...] = jnp.dot(oh.astype(jnp.float32), hb,
                          preferred_element_type=jnp.float32)


@functools.cache
def _tc_layer_kernel(layer):
    return pl.pallas_call(
        _tc_layer_body,
        out_shape=(jax.ShapeDtypeStruct((N, D), jnp.float32),
                   jax.ShapeDtypeStruct((N, 3 * D), jnp.float32),
                   jax.ShapeDtypeStruct((G, D), jnp.float32)),
        in_specs=(
            pl.BlockSpec((NC, N, D), lambda i: (0, 0, 0)),
            pl.BlockSpec(memory_space=pltpu.MemorySpace.HBM),
            pl.BlockSpec((D, D), lambda i: (0, 0)),
            pl.BlockSpec((1, D), lambda i: (0, 0)),
            pl.BlockSpec((D, D), lambda i: (0, 0)),
            pl.BlockSpec((1, D), lambda i: (0, 0)),
            pl.BlockSpec((1, D), lambda i: (0, 0)),
            pl.BlockSpec((1, D), lambda i: (0, 0)),
            pl.BlockSpec((1, N), lambda i: (0, 0)),
        ),
        out_specs=(
            pl.BlockSpec((N, D), lambda i: (0, 0)),
            pl.BlockSpec((N, D), lambda i, new=layer: (0, new)),
            pl.BlockSpec((G, D), lambda i: (0, 0)),
        ),
        input_output_aliases={1: 1},
        grid=(1,),
    )


def kernel(x, edge_index, batch,
           W1_0, b1_0, W2_0, b2_0, gamma_0, beta_0,
           W1_1, b1_1, W2_1, b2_1, gamma_1, beta_1,
           W1_2, b1_2, W2_2, b2_2, gamma_2, beta_2):
    src = edge_index[0]
    dst = edge_index[1].reshape(NW, NCHUNK, K)
    zero = jnp.zeros((N, D), jnp.float32)
    batch2 = batch.reshape(1, N)
    params = [
        (W1_0, b1_0, W2_0, b2_0, gamma_0, beta_0),
        (W1_1, b1_1, W2_1, b2_1, gamma_1, beta_1),
        (W1_2, b1_2, W2_2, b2_2, gamma_2, beta_2),
    ]
    h = x
    xl = jnp.zeros((N, 3 * D), jnp.float32)
    pools = []
    for layer, (W1, b1, W2, b2, gamma, beta) in enumerate(params):
        agg = _sc_agg(h, src, dst, zero)
        h, xl, pool = _tc_layer_kernel(layer)(
            agg, xl, W1, b1.reshape(1, D), W2, b2.reshape(1, D),
            gamma.reshape(1, D), beta.reshape(1, D), batch2)
        pools.append(pool)
    x_global = jnp.concatenate(pools, axis=1)
    return (x_global, xl)


# speedup vs baseline: 3.1199x; 1.2233x over previous
import functools

import jax
import jax.numpy as jnp
from jax import lax
from jax.experimental import pallas as pl
from jax.experimental.pallas import tpu as pltpu
from jax.experimental.pallas import tpu_sc as plsc

N = 10000
E = 320000
D = 128
G = 64
EPS = 1e-5

NC = 2
NS = 16
NW = NC * NS
EW = E // NW
K = 80
NCHUNK = EW // K
RPT = 632
RPT_LAST = N - (NS - 1) * RPT


def _sc_agg_body(h_hbm, src_hbm, dst_hbm, zero_hbm, out_hbm,
                 sidx_v, didx_v, rows0_v, rows1_v, rows2_v, acc_sh,
                 sem0, sem1, sem2, ssem0, ssem1, ssem2):
    c = lax.axis_index("c")
    s = lax.axis_index("s")
    wid = s * NC + c

    row_off = pl.multiple_of(s * RPT, 8)
    cnt_tail = s == NS - 1

    def init_copy(rows, size):
        @pl.when(c == 0)
        def _():
            pltpu.async_copy(h_hbm.at[pl.ds(rows, size)],
                             acc_sh.at[pl.ds(rows, size)], ssem1)

        @pl.when(c != 0)
        def _():
            pltpu.async_copy(zero_hbm.at[pl.ds(rows, size)],
                             acc_sh.at[pl.ds(rows, size)], ssem1)

    @pl.when(cnt_tail)
    def _():
        init_copy((NS - 1) * RPT, RPT_LAST)

    @pl.when(jnp.logical_not(cnt_tail))
    def _():
        init_copy(row_off, RPT)

    pltpu.async_copy(src_hbm.at[pl.ds(pl.multiple_of(wid * EW, 8), EW)],
                     sidx_v, sem1)
    pltpu.async_copy(dst_hbm.at[pl.ds(pl.multiple_of(wid * EW, 8), EW)],
                     didx_v, ssem0)

    def gather(i, buf, sem):
        pltpu.async_copy(h_hbm.at[sidx_v.at[pl.ds(i * K, K)]], buf, sem)

    def wait_gather(i, buf, sem):
        pltpu.make_async_copy(h_hbm.at[sidx_v.at[pl.ds(i * K, K)]],
                              buf, sem).wait()

    def scatter_add(i, buf, sem):
        pltpu.async_copy(buf, acc_sh.at[didx_v.at[pl.ds(i * K, K)]],
                         sem, add=True)

    def wait_scatter(i, buf, sem):
        pltpu.make_async_copy(buf, acc_sh.at[didx_v.at[pl.ds(i * K, K)]],
                              sem).wait()

    pltpu.make_async_copy(src_hbm.at[pl.ds(0, EW)], sidx_v, sem1).wait()
    gather(0, rows0_v, sem0)
    gather(1, rows1_v, sem1)
    gather(2, rows2_v, sem2)

    @pl.when(cnt_tail)
    def _():
        pltpu.make_async_copy(zero_hbm.at[pl.ds((NS - 1) * RPT, RPT_LAST)],
                              acc_sh.at[pl.ds((NS - 1) * RPT, RPT_LAST)],
                              ssem1).wait()

    @pl.when(jnp.logical_not(cnt_tail))
    def _():
        pltpu.make_async_copy(zero_hbm.at[pl.ds(row_off, RPT)],
                              acc_sh.at[pl.ds(row_off, RPT)], ssem1).wait()

    pltpu.make_async_copy(dst_hbm.at[pl.ds(0, EW)], didx_v, ssem0).wait()
    plsc.subcore_barrier()

    def tri(j, carry):
        i0 = 3 * j
        wait_gather(i0, rows0_v, sem0)
        scatter_add(i0, rows0_v, ssem0)
        wait_gather(i0 + 1, rows1_v, sem1)
        scatter_add(i0 + 1, rows1_v, ssem1)
        wait_scatter(i0, rows0_v, ssem0)

        @pl.when(i0 + 3 < NCHUNK)
        def _():
            gather(i0 + 3, rows0_v, sem0)

        wait_gather(i0 + 2, rows2_v, sem2)
        scatter_add(i0 + 2, rows2_v, ssem2)
        wait_scatter(i0 + 1, rows1_v, ssem1)

        @pl.when(i0 + 4 < NCHUNK)
        def _():
            gather(i0 + 4, rows1_v, sem1)

        wait_scatter(i0 + 2, rows2_v, ssem2)

        @pl.when(i0 + 5 < NCHUNK)
        def _():
            gather(i0 + 5, rows2_v, sem2)

        return carry

    lax.fori_loop(0, NCHUNK // 3, tri, 0)
    wait_gather(NCHUNK - 2, rows0_v, sem0)
    pltpu.sync_copy(rows0_v, acc_sh.at[didx_v.at[pl.ds((NCHUNK - 2) * K, K)]],
                    add=True)
    wait_gather(NCHUNK - 1, rows1_v, sem1)
    pltpu.sync_copy(rows1_v, acc_sh.at[didx_v.at[pl.ds((NCHUNK - 1) * K, K)]],
                    add=True)

    plsc.subcore_barrier()

    @pl.when(s < NS - 1)
    def _():
        pltpu.sync_copy(acc_sh.at[pl.ds(row_off, RPT)],
                        out_hbm.at[c, pl.ds(row_off, RPT)])

    @pl.when(s == NS - 1)
    def _():
        pltpu.sync_copy(acc_sh.at[pl.ds((NS - 1) * RPT, RPT_LAST)],
                        out_hbm.at[c, pl.ds((NS - 1) * RPT, RPT_LAST)])


@functools.cache
def _sc_agg_kernel():
    return pl.kernel(
        _sc_agg_body,
        out_type=jax.ShapeDtypeStruct((NC, N, D), jnp.float32),
        mesh=plsc.VectorSubcoreMesh(core_axis_name="c", subcore_axis_name="s",
                                    num_cores=NC, num_subcores=NS),
        scratch_types=[
            pltpu.VMEM((EW,), jnp.int32),
            pltpu.VMEM((EW,), jnp.int32),
            pltpu.VMEM((K, D), jnp.float32),
            pltpu.VMEM((K, D), jnp.float32),
            pltpu.VMEM((K, D), jnp.float32),
            pltpu.VMEM_SHARED((N, D), jnp.float32),
            pltpu.SemaphoreType.DMA,
            pltpu.SemaphoreType.DMA,
            pltpu.SemaphoreType.DMA,
            pltpu.SemaphoreType.DMA,
            pltpu.SemaphoreType.DMA,
            pltpu.SemaphoreType.DMA,
        ],
    )


def _sc_agg(h, src, dst, zero):
    return _sc_agg_kernel()(h, src, dst, zero)


def _tc_layer_body(agg_ref, xl_in_ref, w1_ref, b1_ref, w2_ref, b2_ref,
                   g_ref, be_ref, batch_ref, ho_ref, xl_ref, po_ref):
    h = agg_ref[0] + agg_ref[1]
    h = jnp.dot(h, w1_ref[...], preferred_element_type=jnp.float32)
    h = h + b1_ref[...]
    h = jnp.where(h > 0, h, 0.01 * h)
    h = jnp.dot(h, w2_ref[...], preferred_element_type=jnp.float32)
    h = h + b2_ref[...]
    h = jnp.where(h > 0, h, 0.01 * h)
    m = jnp.mean(h, axis=1, keepdims=True)
    hc = h - m
    v = jnp.mean(hc * hc, axis=1, keepdims=True)
    hn = hc * lax.rsqrt(v + EPS)
    bm = jnp.mean(hn, axis=0, keepdims=True)
    hb = hn - bm
    bv = jnp.mean(hb * hb, axis=0, keepdims=True)
    hb = hb * lax.rsqrt(bv + EPS) * g_ref[...] + be_ref[...]
    ho_ref[...] = hb
    xl_ref[...] = hb
    oh = (batch_ref[...] == lax.broadcasted_iota(jnp.int32, (G, 1), 0))
    po_ref[...] = jnp.dot(oh.astype(jnp.float32), hb,
                          preferred_element_type=jnp.float32)


@functools.cache
def _tc_layer_kernel(layer):
    return pl.pallas_call(
        _tc_layer_body,
        out_shape=(jax.ShapeDtypeStruct((N, D), jnp.float32),
                   jax.ShapeDtypeStruct((N, 3 * D), jnp.float32),
                   jax.ShapeDtypeStruct((G, D), jnp.float32)),
        in_specs=(
            pl.BlockSpec((NC, N, D), lambda i: (0, 0, 0)),
            pl.BlockSpec(memory_space=pltpu.MemorySpace.HBM),
            pl.BlockSpec((D, D), lambda i: (0, 0)),
            pl.BlockSpec((1, D), lambda i: (0, 0)),
            pl.BlockSpec((D, D), lambda i: (0, 0)),
            pl.BlockSpec((1, D), lambda i: (0, 0)),
            pl.BlockSpec((1, D), lambda i: (0, 0)),
            pl.BlockSpec((1, D), lambda i: (0, 0)),
            pl.BlockSpec((1, N), lambda i: (0, 0)),
        ),
        out_specs=(
            pl.BlockSpec((N, D), lambda i: (0, 0)),
            pl.BlockSpec((N, D), lambda i, new=layer: (0, new)),
            pl.BlockSpec((G, D), lambda i: (0, 0)),
        ),
        input_output_aliases={1: 1},
        grid=(1,),
    )


def kernel(x, edge_index, batch,
           W1_0, b1_0, W2_0, b2_0, gamma_0, beta_0,
           W1_1, b1_1, W2_1, b2_1, gamma_1, beta_1,
           W1_2, b1_2, W2_2, b2_2, gamma_2, beta_2):
    src = edge_index[0]
    dst = edge_index[1]
    zero = jnp.zeros((N, D), jnp.float32)
    batch2 = batch.reshape(1, N)
    params = [
        (W1_0, b1_0, W2_0, b2_0, gamma_0, beta_0),
        (W1_1, b1_1, W2_1, b2_1, gamma_1, beta_1),
        (W1_2, b1_2, W2_2, b2_2, gamma_2, beta_2),
    ]
    h = x
    xl = jnp.zeros((N, 3 * D), jnp.float32)
    pools = []
    for layer, (W1, b1, W2, b2, gamma, beta) in enumerate(params):
        agg = _sc_agg(h, src, dst, zero)
        h, xl, pool = _tc_layer_kernel(layer)(
            agg, xl, W1, b1.reshape(1, D), W2, b2.reshape(1, D),
            gamma.reshape(1, D), beta.reshape(1, D), batch2)
        pools.append(pool)
    x_global = jnp.concatenate(pools, axis=1)
    return (x_global, xl)
